# Initial kernel scaffold; baseline (speedup 1.0000x reference)
import jax
import jax.numpy as jnp
from jax.experimental import pallas as pl


def kernel(x, edge_index, Wn0, bn0, Ws0, Wn1, bn1, Ws1, Wn2, bn2, Ws2, Wn3, bn3, Ws3, Wfc, bfc):
    raise NotImplementedError("write your pallas kernel here")


# speedup vs baseline: 11.1163x; 11.1163x over previous
import functools

import jax
import jax.numpy as jnp
from jax import lax
from jax.experimental import pallas as pl
from jax.experimental.pallas import tpu as pltpu
from jax.experimental.pallas import tpu_sc as plsc

N = 10000
E = 160000
D = 256
DIM = 32
C = 10

NC = 2
NS = 16
NW = NC * NS

CH = 1000
NCHUNKS = E // CH
CPW = NCHUNKS // NW
RPT = N // NS

_MESH = plsc.VectorSubcoreMesh(core_axis_name="c", subcore_axis_name="s")


@functools.partial(
    pl.kernel,
    out_type=jax.ShapeDtypeStruct((NC * N, DIM), jnp.float32),
    mesh=_MESH,
    scratch_types=[
        pltpu.VMEM((2, CH), jnp.int32),
        pltpu.VMEM((CH, DIM), jnp.float32),
        pltpu.VMEM_SHARED((N, DIM), jnp.float32),
        pltpu.SemaphoreType.DMA,
    ],
    compiler_params=pltpu.CompilerParams(use_tc_tiling_on_sc=False),
)
def _segsum_sc(m_hbm, ei_hbm, zeros_hbm, out_hbm,
               idx_v, rows_v, agg_sh, sem):
    c = lax.axis_index("c")
    s = lax.axis_index("s")
    wid = s * NC + c

    pltpu.sync_copy(zeros_hbm.at[pl.ds(s * RPT, RPT)],
                    agg_sh.at[pl.ds(s * RPT, RPT)])
    plsc.subcore_barrier()

    def body(g, carry):
        off = pl.multiple_of((g * NW + wid) * CH, CH)
        pltpu.sync_copy(ei_hbm.at[:, pl.ds(off, CH)], idx_v)
        pltpu.async_copy(m_hbm.at[idx_v.at[0]], rows_v, sem).wait()
        pltpu.sync_copy(rows_v, agg_sh.at[idx_v.at[1]], add=True)
        return carry

    lax.fori_loop(0, CPW, body, 0)
    plsc.subcore_barrier()

    pltpu.sync_copy(agg_sh.at[pl.ds(s * RPT, RPT)],
                    out_hbm.at[pl.ds(c * N + s * RPT, RPT)])


_RB = 2000
_G = N // _RB


def _mm_kernel(h_ref, wn_ref, ws_ref, m_ref, slf_ref):
    h = h_ref[...]
    m_ref[...] = jnp.dot(h, wn_ref[...], preferred_element_type=jnp.float32)
    slf_ref[...] = jnp.dot(h, ws_ref[...], preferred_element_type=jnp.float32)


def _mm(h, wn, ws):
    d_in = h.shape[1]
    return pl.pallas_call(
        _mm_kernel,
        grid=(_G,),
        in_specs=[
            pl.BlockSpec((_RB, d_in), lambda i: (i, 0)),
            pl.BlockSpec((d_in, DIM), lambda i: (0, 0)),
            pl.BlockSpec((d_in, DIM), lambda i: (0, 0)),
        ],
        out_specs=[
            pl.BlockSpec((_RB, DIM), lambda i: (i, 0)),
            pl.BlockSpec((_RB, DIM), lambda i: (i, 0)),
        ],
        out_shape=[
            jax.ShapeDtypeStruct((N, DIM), jnp.float32),
            jax.ShapeDtypeStruct((N, DIM), jnp.float32),
        ],
    )(h, wn, ws)


def _combine_mm_kernel(a0_ref, a1_ref, bn_ref, slf_ref, wn_ref, ws_ref,
                       h_ref, m_ref, slf2_ref):
    h = jnp.maximum(a0_ref[...] + a1_ref[...] + bn_ref[...] + slf_ref[...], 0.0)
    h_ref[...] = h
    m_ref[...] = jnp.dot(h, wn_ref[...], preferred_element_type=jnp.float32)
    slf2_ref[...] = jnp.dot(h, ws_ref[...], preferred_element_type=jnp.float32)


def _combine_mm(aggsc, bn, slf, wn, ws):
    a0 = aggsc[:N]
    a1 = aggsc[N:]
    return pl.pallas_call(
        _combine_mm_kernel,
        grid=(_G,),
        in_specs=[
            pl.BlockSpec((_RB, DIM), lambda i: (i, 0)),
            pl.BlockSpec((_RB, DIM), lambda i: (i, 0)),
            pl.BlockSpec((1, DIM), lambda i: (0, 0)),
            pl.BlockSpec((_RB, DIM), lambda i: (i, 0)),
            pl.BlockSpec((DIM, DIM), lambda i: (0, 0)),
            pl.BlockSpec((DIM, DIM), lambda i: (0, 0)),
        ],
        out_specs=[
            pl.BlockSpec((_RB, DIM), lambda i: (i, 0)),
            pl.BlockSpec((_RB, DIM), lambda i: (i, 0)),
            pl.BlockSpec((_RB, DIM), lambda i: (i, 0)),
        ],
        out_shape=[
            jax.ShapeDtypeStruct((N, DIM), jnp.float32),
            jax.ShapeDtypeStruct((N, DIM), jnp.float32),
            jax.ShapeDtypeStruct((N, DIM), jnp.float32),
        ],
    )(a0, a1, bn.reshape(1, DIM), slf, wn, ws)


def _final_kernel(x_ref, h1_ref, h2_ref, h3_ref, a0_ref, a1_ref, bn_ref,
                  slf_ref, wfc_ref, bfc_ref, out_ref,
                  accx_ref, acc1_ref, acc2_ref, acc3_ref, acc4_ref):
    i = pl.program_id(0)

    @pl.when(i == 0)
    def _():
        accx_ref[...] = jnp.zeros_like(accx_ref)
        acc1_ref[...] = jnp.zeros_like(acc1_ref)
        acc2_ref[...] = jnp.zeros_like(acc2_ref)
        acc3_ref[...] = jnp.zeros_like(acc3_ref)
        acc4_ref[...] = jnp.zeros_like(acc4_ref)

    h4 = jnp.maximum(a0_ref[...] + a1_ref[...] + bn_ref[...] + slf_ref[...],
                     0.0)
    accx_ref[...] += jnp.sum(x_ref[...], axis=0, keepdims=True)
    acc1_ref[...] += jnp.sum(h1_ref[...], axis=0, keepdims=True)
    acc2_ref[...] += jnp.sum(h2_ref[...], axis=0, keepdims=True)
    acc3_ref[...] += jnp.sum(h3_ref[...], axis=0, keepdims=True)
    acc4_ref[...] += jnp.sum(h4, axis=0, keepdims=True)

    @pl.when(i == _G - 1)
    def _():
        inv_n = 1.0 / N
        dot = lambda a, b: jnp.dot(a, b, preferred_element_type=jnp.float32)
        logits = (dot(accx_ref[...] * inv_n, wfc_ref[pl.ds(0, D), :])
                  + dot(acc1_ref[...] * inv_n, wfc_ref[pl.ds(D, DIM), :])
                  + dot(acc2_ref[...] * inv_n, wfc_ref[pl.ds(D + DIM, DIM), :])
                  + dot(acc3_ref[...] * inv_n,
                        wfc_ref[pl.ds(D + 2 * DIM, DIM), :])
                  + dot(acc4_ref[...] * inv_n,
                        wfc_ref[pl.ds(D + 3 * DIM, DIM), :])
                  + bfc_ref[...])
        mx = jnp.max(logits, axis=1, keepdims=True)
        z = logits - mx
        out_ref[...] = z - jnp.log(jnp.sum(jnp.exp(z), axis=1, keepdims=True))


def _final(x, h1, h2, h3, aggsc, bn, slf, wfc, bfc):
    d_cat = D + 4 * DIM
    a0 = aggsc[:N]
    a1 = aggsc[N:]
    return pl.pallas_call(
        _final_kernel,
        grid=(_G,),
        in_specs=[
            pl.BlockSpec((_RB, D), lambda i: (i, 0)),
            pl.BlockSpec((_RB, DIM), lambda i: (i, 0)),
            pl.BlockSpec((_RB, DIM), lambda i: (i, 0)),
            pl.BlockSpec((_RB, DIM), lambda i: (i, 0)),
            pl.BlockSpec((_RB, DIM), lambda i: (i, 0)),
            pl.BlockSpec((_RB, DIM), lambda i: (i, 0)),
            pl.BlockSpec((1, DIM), lambda i: (0, 0)),
            pl.BlockSpec((_RB, DIM), lambda i: (i, 0)),
            pl.BlockSpec((d_cat, C), lambda i: (0, 0)),
            pl.BlockSpec((1, C), lambda i: (0, 0)),
        ],
        out_specs=pl.BlockSpec((1, C), lambda i: (0, 0)),
        out_shape=jax.ShapeDtypeStruct((1, C), jnp.float32),
        scratch_shapes=[
            pltpu.VMEM((1, D), jnp.float32),
            pltpu.VMEM((1, DIM), jnp.float32),
            pltpu.VMEM((1, DIM), jnp.float32),
            pltpu.VMEM((1, DIM), jnp.float32),
            pltpu.VMEM((1, DIM), jnp.float32),
        ],
    )(x, h1, h2, h3, a0, a1, bn.reshape(1, DIM), slf, wfc,
      bfc.reshape(1, C))


def kernel(x, edge_index, Wn0, bn0, Ws0, Wn1, bn1, Ws1, Wn2, bn2, Ws2,
           Wn3, bn3, Ws3, Wfc, bfc):
    zeros = jnp.zeros((N, DIM), jnp.float32)

    m, slf = _mm(x, Wn0, Ws0)
    aggsc = _segsum_sc(m, edge_index, zeros)

    h = [None] * 4
    params = [(Wn1, bn0, Ws1), (Wn2, bn1, Ws2), (Wn3, bn2, Ws3)]
    for l, (wn_next, bn, ws_next) in enumerate(params):
        h[l], m, slf = _combine_mm(aggsc, bn, slf, wn_next, ws_next)
        aggsc = _segsum_sc(m, edge_index, zeros)

    return _final(x, h[0], h[1], h[2], aggsc, bn3, slf, Wfc, bfc)
